# Initial kernel scaffold; baseline (speedup 1.0000x reference)
import functools

import jax
import jax.numpy as jnp
from jax import lax
from jax.experimental import pallas as pl
from jax.experimental.pallas import tpu as pltpu
from jax.experimental.pallas import tpu_sc as plsc

DIM = 64
NCTX = 4
NNEG = 5
NC, NS, L = 2, 16, 16
NW = NC * NS
CHUNK = 128


def _sc_scores_call(B):
    bpw = B // NW
    nchunk = bpw // CHUNK
    mesh = plsc.VectorSubcoreMesh(core_axis_name="c", subcore_axis_name="s")

    @functools.partial(
        pl.kernel,
        out_type=(jax.ShapeDtypeStruct((B,), jnp.float32),
                  jax.ShapeDtypeStruct((B,), jnp.float32)),
        mesh=mesh,
        scratch_types=[
            pltpu.VMEM((CHUNK,), jnp.int32),
            pltpu.VMEM((NCTX, CHUNK), jnp.int32),
            pltpu.VMEM((NNEG, CHUNK), jnp.int32),
            pltpu.VMEM((CHUNK, DIM), jnp.float32),
            pltpu.VMEM((NCTX * CHUNK, DIM), jnp.float32),
            pltpu.VMEM((NNEG * CHUNK, DIM), jnp.float32),
            pltpu.VMEM((CHUNK,), jnp.float32),
            pltpu.VMEM((CHUNK,), jnp.float32),
            pltpu.SemaphoreType.DMA,
        ],
    )
    def sc_scores(cen_hbm, ctx_hbm, neg_hbm, w_hbm, pos_hbm, negs_hbm,
                  cidx, xidx, nidx, crows, xrows, nrows, pos_v, neg_v, sem):
        wid = lax.axis_index("s") * NC + lax.axis_index("c")

        for g in range(nchunk):
            cb = wid * bpw + g * CHUNK
            crow = wid * (bpw // CHUNK) + g

            pltpu.sync_copy(cen_hbm.at[crow], cidx)
            pltpu.sync_copy(ctx_hbm.at[pl.ds(crow * NCTX, NCTX)], xidx)
            pltpu.sync_copy(neg_hbm.at[pl.ds(crow * NNEG, NNEG)], nidx)

            copies = [pltpu.async_copy(w_hbm.at[cidx], crows, sem)]
            for j in range(NCTX):
                copies.append(pltpu.async_copy(
                    w_hbm.at[xidx.at[j]],
                    xrows.at[pl.ds(j * CHUNK, CHUNK)], sem))
            for j in range(NNEG):
                copies.append(pltpu.async_copy(
                    w_hbm.at[nidx.at[j]],
                    nrows.at[pl.ds(j * CHUNK, CHUNK)], sem))
            for c in copies:
                c.wait()

            def body(bl, carry):
                pos_acc = jnp.zeros((L,), jnp.float32)
                neg_acc = jnp.zeros((L,), jnp.float32)
                for j in range(DIM // L):
                    u = crows[bl, pl.ds(j * L, L)]
                    xs = xrows[NCTX * bl, pl.ds(j * L, L)]
                    for k in range(1, NCTX):
                        xs = xs + xrows[NCTX * bl + k, pl.ds(j * L, L)]
                    ns = nrows[NNEG * bl, pl.ds(j * L, L)]
                    for k in range(1, NNEG):
                        ns = ns + nrows[NNEG * bl + k, pl.ds(j * L, L)]
                    pos_acc = pos_acc + u * xs
                    neg_acc = neg_acc + u * ns
                pos_v[bl] = jnp.sum(pos_acc) * (1.0 / NCTX)
                neg_v[bl] = jnp.sum(neg_acc) * (1.0 / NNEG)
                return carry

            lax.fori_loop(0, CHUNK, body, 0, unroll=2)

            pltpu.sync_copy(pos_v, pos_hbm.at[pl.ds(cb, CHUNK)])
            pltpu.sync_copy(neg_v, negs_hbm.at[pl.ds(cb, CHUNK)])

    return sc_scores


def _tail_body(pos_ref, neg_ref, out_ref):
    p = pos_ref[...]
    n = -neg_ref[...]
    lsp = jnp.minimum(p, 0.0) - jnp.log(1.0 + jnp.exp(-jnp.abs(p)))
    lsn = jnp.minimum(n, 0.0) - jnp.log(1.0 + jnp.exp(-jnp.abs(n)))
    b = pos_ref.shape[0] * pos_ref.shape[1]
    out_ref[0, 0] = -(jnp.sum(lsp) + jnp.sum(lsn)) / b


def kernel(centers, context, neg_context, W):
    B = centers.shape[0]
    cen2d = centers.reshape(B // CHUNK, CHUNK)
    ctx2d = context.reshape(B * NCTX // CHUNK, CHUNK)
    neg2d = neg_context.reshape(B * NNEG // CHUNK, CHUNK)

    pos, neg = _sc_scores_call(B)(cen2d, ctx2d, neg2d, W)

    rows = B // 128
    loss = pl.pallas_call(
        _tail_body,
        out_shape=jax.ShapeDtypeStruct((1, 1), jnp.float32),
    )(pos.reshape(rows, 128), neg.reshape(rows, 128))
    return loss[0, 0]


# speedup vs baseline: 4.1372x; 4.1372x over previous
import functools

import jax
import jax.numpy as jnp
from jax import lax
from jax.experimental import pallas as pl
from jax.experimental.pallas import tpu as pltpu
from jax.experimental.pallas import tpu_sc as plsc

DIM = 64
NCTX = 4
NNEG = 5
NC, NS, L = 2, 16, 16
NW = NC * NS
CHUNK = 64


def _sc_scores_call(B):
    bpw = B // NW
    nchunk = bpw // CHUNK
    mesh = plsc.VectorSubcoreMesh(core_axis_name="c", subcore_axis_name="s")

    @functools.partial(
        pl.kernel,
        out_type=(jax.ShapeDtypeStruct((B,), jnp.float32),
                  jax.ShapeDtypeStruct((B,), jnp.float32)),
        mesh=mesh,
        compiler_params=pltpu.CompilerParams(needs_layout_passes=False),
        scratch_types=[
            pltpu.VMEM((CHUNK,), jnp.int32),
            pltpu.VMEM((CHUNK,), jnp.int32),
            pltpu.VMEM((CHUNK,), jnp.int32),
            pltpu.VMEM((NCTX * CHUNK,), jnp.int32),
            pltpu.VMEM((NCTX * CHUNK,), jnp.int32),
            pltpu.VMEM((NCTX * CHUNK,), jnp.int32),
            pltpu.VMEM((NNEG * CHUNK,), jnp.int32),
            pltpu.VMEM((NNEG * CHUNK,), jnp.int32),
            pltpu.VMEM((NNEG * CHUNK,), jnp.int32),
            pltpu.VMEM((CHUNK, 2 * DIM), jnp.float32),
            pltpu.VMEM((NCTX * CHUNK, 2 * DIM), jnp.float32),
            pltpu.VMEM((NNEG * CHUNK, 2 * DIM), jnp.float32),
            pltpu.VMEM((CHUNK,), jnp.float32),
            pltpu.VMEM((CHUNK,), jnp.float32),
            pltpu.SemaphoreType.DMA,
        ],
    )
    def sc_scores(cen_hbm, ctx_hbm, neg_hbm, w2_hbm, pos_hbm, negs_hbm,
                  cidx, cpair, choff, xidx, xpair, xhoff, nidx, npair, nhoff,
                  crows, xrows, nrows, pos_v, neg_v, sem):
        wid = lax.axis_index("s") * NC + lax.axis_index("c")
        lanes = lax.iota(jnp.int32, L)

        def split(src, dst_pair, dst_hoff, n):
            for m in range(n // L):
                v = src[pl.ds(m * L, L)]
                dst_pair[pl.ds(m * L, L)] = lax.shift_right_logical(v, 1)
                dst_hoff[pl.ds(m * L, L)] = lax.shift_left(v & 1, 6)

        def chunk_body(g, carry):
            cb = wid * bpw + g * CHUNK

            pltpu.sync_copy(cen_hbm.at[pl.ds(cb, CHUNK)], cidx)
            pltpu.sync_copy(ctx_hbm.at[pl.ds(cb * NCTX, NCTX * CHUNK)], xidx)
            pltpu.sync_copy(neg_hbm.at[pl.ds(cb * NNEG, NNEG * CHUNK)], nidx)

            split(cidx, cpair, choff, CHUNK)
            split(xidx, xpair, xhoff, NCTX * CHUNK)
            split(nidx, npair, nhoff, NNEG * CHUNK)

            copies = [pltpu.async_copy(w2_hbm.at[cpair], crows, sem)]
            for j in range(NCTX):
                copies.append(pltpu.async_copy(
                    w2_hbm.at[xpair.at[pl.ds(j * CHUNK, CHUNK)]],
                    xrows.at[pl.ds(j * CHUNK, CHUNK)], sem))
            for j in range(NNEG):
                copies.append(pltpu.async_copy(
                    w2_hbm.at[npair.at[pl.ds(j * CHUNK, CHUNK)]],
                    nrows.at[pl.ds(j * CHUNK, CHUNK)], sem))
            for c in copies:
                c.wait()

            def group_body(t, carry2):
                bvec = t * L + lanes
                uoff = choff[pl.ds(t * L, L)]
                xrow = [NCTX * bvec + k for k in range(NCTX)]
                nrow = [NNEG * bvec + k for k in range(NNEG)]
                xoff = [plsc.load_gather(xhoff, [xrow[k]])
                        for k in range(NCTX)]
                noff = [plsc.load_gather(nhoff, [nrow[k]])
                        for k in range(NNEG)]
                pos_acc = jnp.zeros((L,), jnp.float32)
                neg_acc = jnp.zeros((L,), jnp.float32)
                for d in range(DIM):
                    dv = (d + lanes) & (DIM - 1)
                    u = plsc.load_gather(crows, [bvec, uoff + dv])
                    xs = plsc.load_gather(xrows, [xrow[0], xoff[0] + dv])
                    for k in range(1, NCTX):
                        xs = xs + plsc.load_gather(
                            xrows, [xrow[k], xoff[k] + dv])
                    ns = plsc.load_gather(nrows, [nrow[0], noff[0] + dv])
                    for k in range(1, NNEG):
                        ns = ns + plsc.load_gather(
                            nrows, [nrow[k], noff[k] + dv])
                    pos_acc = pos_acc + u * xs
                    neg_acc = neg_acc + u * ns
                pos_v[pl.ds(t * L, L)] = pos_acc * (1.0 / NCTX)
                neg_v[pl.ds(t * L, L)] = neg_acc * (1.0 / NNEG)
                return carry2

            lax.fori_loop(0, CHUNK // L, group_body, 0)

            pltpu.sync_copy(pos_v, pos_hbm.at[pl.ds(cb, CHUNK)])
            pltpu.sync_copy(neg_v, negs_hbm.at[pl.ds(cb, CHUNK)])
            return carry

        lax.fori_loop(0, nchunk, chunk_body, 0)

    return sc_scores


def _tail_body(pos_ref, neg_ref, out_ref):
    p = pos_ref[...]
    n = -neg_ref[...]
    lsp = jnp.minimum(p, 0.0) - jnp.log(1.0 + jnp.exp(-jnp.abs(p)))
    lsn = jnp.minimum(n, 0.0) - jnp.log(1.0 + jnp.exp(-jnp.abs(n)))
    b = pos_ref.shape[0] * pos_ref.shape[1]
    out_ref[...] = jnp.full((1, 1), -(jnp.sum(lsp) + jnp.sum(lsn)) / b,
                            jnp.float32)


def kernel(centers, context, neg_context, W):
    B = centers.shape[0]
    V = W.shape[0]
    w2 = W.reshape(V // 2, 2 * DIM)
    pos, neg = _sc_scores_call(B)(
        centers, context.reshape(-1), neg_context.reshape(-1), w2)

    rows = B // 128
    loss = pl.pallas_call(
        _tail_body,
        out_shape=jax.ShapeDtypeStruct((1, 1), jnp.float32),
    )(pos.reshape(rows, 128), neg.reshape(rows, 128))
    return loss[0, 0]


# speedup vs baseline: 4.4007x; 1.0637x over previous
import functools

import jax
import jax.numpy as jnp
from jax import lax
from jax.experimental import pallas as pl
from jax.experimental.pallas import tpu as pltpu
from jax.experimental.pallas import tpu_sc as plsc

DIM = 64
NCTX = 4
NNEG = 5
NC, NS, L = 2, 16, 16
NW = NC * NS
CHUNK = 128


def _sc_scores_call(B):
    bpw = B // NW
    nchunk = bpw // CHUNK
    mesh = plsc.VectorSubcoreMesh(core_axis_name="c", subcore_axis_name="s")

    @functools.partial(
        pl.kernel,
        out_type=(jax.ShapeDtypeStruct((B,), jnp.float32),
                  jax.ShapeDtypeStruct((B,), jnp.float32)),
        mesh=mesh,
        compiler_params=pltpu.CompilerParams(needs_layout_passes=False,
                                             use_tc_tiling_on_sc=False),
        scratch_types=[
            pltpu.VMEM((CHUNK,), jnp.int32),
            pltpu.VMEM((NCTX * CHUNK,), jnp.int32),
            pltpu.VMEM((NNEG * CHUNK,), jnp.int32),
            pltpu.VMEM((CHUNK, DIM), jnp.float32),
            pltpu.VMEM((NCTX * CHUNK, DIM), jnp.float32),
            pltpu.VMEM((NNEG * CHUNK, DIM), jnp.float32),
            pltpu.VMEM((CHUNK,), jnp.float32),
            pltpu.VMEM((CHUNK,), jnp.float32),
            pltpu.SemaphoreType.DMA,
        ],
    )
    def sc_scores(cen_hbm, ctx_hbm, neg_hbm, w_hbm, pos_hbm, negs_hbm,
                  cidx, xidx, nidx, crows, xrows, nrows, pos_v, neg_v, sem):
        wid = lax.axis_index("s") * NC + lax.axis_index("c")
        lanes = lax.iota(jnp.int32, L)

        def chunk_body(g, carry):
            cb = wid * bpw + g * CHUNK

            pltpu.sync_copy(cen_hbm.at[pl.ds(cb, CHUNK)], cidx)
            pltpu.sync_copy(ctx_hbm.at[pl.ds(cb * NCTX, NCTX * CHUNK)], xidx)
            pltpu.sync_copy(neg_hbm.at[pl.ds(cb * NNEG, NNEG * CHUNK)], nidx)

            copies = [pltpu.async_copy(w_hbm.at[cidx], crows, sem)]
            for j in range(NCTX):
                copies.append(pltpu.async_copy(
                    w_hbm.at[xidx.at[pl.ds(j * CHUNK, CHUNK)]],
                    xrows.at[pl.ds(j * CHUNK, CHUNK)], sem))
            for j in range(NNEG):
                copies.append(pltpu.async_copy(
                    w_hbm.at[nidx.at[pl.ds(j * CHUNK, CHUNK)]],
                    nrows.at[pl.ds(j * CHUNK, CHUNK)], sem))
            for c in copies:
                c.wait()

            def group_body(t, carry2):
                bvec = t * L + lanes
                xrow = [NCTX * bvec + k for k in range(NCTX)]
                nrow = [NNEG * bvec + k for k in range(NNEG)]
                pos_acc = jnp.zeros((L,), jnp.float32)
                neg_acc = jnp.zeros((L,), jnp.float32)
                for d in range(DIM):
                    dv = (d + lanes) & (DIM - 1)
                    u = plsc.load_gather(crows, [bvec, dv])
                    xs = plsc.load_gather(xrows, [xrow[0], dv])
                    for k in range(1, NCTX):
                        xs = xs + plsc.load_gather(xrows, [xrow[k], dv])
                    ns = plsc.load_gather(nrows, [nrow[0], dv])
                    for k in range(1, NNEG):
                        ns = ns + plsc.load_gather(nrows, [nrow[k], dv])
                    pos_acc = pos_acc + u * xs
                    neg_acc = neg_acc + u * ns
                pos_v[pl.ds(t * L, L)] = pos_acc * (1.0 / NCTX)
                neg_v[pl.ds(t * L, L)] = neg_acc * (1.0 / NNEG)
                return carry2

            lax.fori_loop(0, CHUNK // L, group_body, 0)

            pltpu.sync_copy(pos_v, pos_hbm.at[pl.ds(cb, CHUNK)])
            pltpu.sync_copy(neg_v, negs_hbm.at[pl.ds(cb, CHUNK)])
            return carry

        lax.fori_loop(0, nchunk, chunk_body, 0)

    return sc_scores


def _tail_body(pos_ref, neg_ref, out_ref):
    p = pos_ref[...]
    n = -neg_ref[...]
    lsp = jnp.minimum(p, 0.0) - jnp.log(1.0 + jnp.exp(-jnp.abs(p)))
    lsn = jnp.minimum(n, 0.0) - jnp.log(1.0 + jnp.exp(-jnp.abs(n)))
    b = pos_ref.shape[0] * pos_ref.shape[1]
    out_ref[...] = jnp.full((1, 1), -(jnp.sum(lsp) + jnp.sum(lsn)) / b,
                            jnp.float32)


def kernel(centers, context, neg_context, W):
    B = centers.shape[0]
    pos, neg = _sc_scores_call(B)(
        centers, context.reshape(-1), neg_context.reshape(-1), W)

    rows = B // 128
    loss = pl.pallas_call(
        _tail_body,
        out_shape=jax.ShapeDtypeStruct((1, 1), jnp.float32),
    )(pos.reshape(rows, 128), neg.reshape(rows, 128))
    return loss[0, 0]
